# Initial kernel scaffold; baseline (speedup 1.0000x reference)
import jax
import jax.numpy as jnp
from jax.experimental import pallas as pl


def kernel(x, edge_index, W1, a_src1, a_dst1, b1, W2, a_src2, a_dst2, b2):
    raise NotImplementedError("write your pallas kernel here")


# speedup vs baseline: 117.2848x; 117.2848x over previous
import functools

import jax
import jax.numpy as jnp
from jax import lax
from jax.experimental import pallas as pl
from jax.experimental.pallas import tpu as pltpu
from jax.experimental.pallas import tpu_sc as plsc

_NC = 2
_NS = 16
_LANES = 16
_K = 128


def _leaky(v):
    return jnp.where(v > 0, v, 0.2 * v)


def _pro1_body(x_ref, w_ref, s_ref, d_ref, h_ref, as_ref, ad_ref, m_ref):
    i = pl.program_id(0)
    x = x_ref[...]
    h0 = jnp.dot(x, w_ref[:, :16], preferred_element_type=jnp.float32)
    h1 = jnp.dot(x, w_ref[:, 16:], preferred_element_type=jnp.float32)
    as0 = jnp.sum(h0 * s_ref[0, :], axis=-1)
    as1 = jnp.sum(h1 * s_ref[1, :], axis=-1)
    ad0 = jnp.sum(h0 * d_ref[0, :], axis=-1)
    ad1 = jnp.sum(h1 * d_ref[1, :], axis=-1)
    h_ref[...] = jnp.stack([h0, h1])
    as_ref[...] = jnp.stack([as0, as1])
    ad_ref[...] = jnp.stack([ad0, ad1])
    part = jnp.stack([jnp.max(as0), jnp.max(as1), jnp.max(ad0), jnp.max(ad1)])
    part = jnp.concatenate([part, part]).reshape(1, 8)

    @pl.when(i == 0)
    def _():
        m_ref[...] = part

    @pl.when(i != 0)
    def _():
        m_ref[...] = jnp.maximum(m_ref[...], part)


def _mid_body(acc_ref, den_ref, b1_ref, w2_ref, s2_ref, d2_ref,
              h_ref, as_ref, ad_ref, m_ref):
    i = pl.program_id(0)
    b = acc_ref.shape[1]
    g0 = acc_ref[0] / den_ref[0][:, None]
    g1 = acc_ref[1] / den_ref[1][:, None]
    xh = jnp.concatenate([g0, g1], axis=-1) + b1_ref[0, :]
    xh = jnp.maximum(xh, 0.0)
    h2 = jnp.dot(xh, w2_ref[...], preferred_element_type=jnp.float32)
    h20 = h2[:, :7]
    h21 = h2[:, 7:14]
    as0 = jnp.sum(h20 * s2_ref[0, :], axis=-1)
    as1 = jnp.sum(h21 * s2_ref[1, :], axis=-1)
    ad0 = jnp.sum(h20 * d2_ref[0, :], axis=-1)
    ad1 = jnp.sum(h21 * d2_ref[1, :], axis=-1)
    z = jnp.zeros((b, 9), jnp.float32)
    h_ref[...] = jnp.stack([jnp.concatenate([h20, z], axis=-1),
                            jnp.concatenate([h21, z], axis=-1)])
    as_ref[...] = jnp.stack([as0, as1])
    ad_ref[...] = jnp.stack([ad0, ad1])
    part = jnp.stack([jnp.max(as0), jnp.max(as1), jnp.max(ad0), jnp.max(ad1)])
    part = jnp.concatenate([part, part]).reshape(1, 8)

    @pl.when(i == 0)
    def _():
        m_ref[...] = part

    @pl.when(i != 0)
    def _():
        m_ref[...] = jnp.maximum(m_ref[...], part)


def _init_body(h_ref, as_ref, ad_ref, m_ref, acc_ref, den_ref, m16_ref):
    i = pl.program_id(0)
    m0 = _leaky(m_ref[0, 0] + m_ref[0, 2])
    m1 = _leaky(m_ref[0, 1] + m_ref[0, 3])
    w0 = jnp.exp(_leaky(as_ref[0] + ad_ref[0]) - m0)
    w1 = jnp.exp(_leaky(as_ref[1] + ad_ref[1]) - m1)
    acc_ref[...] = jnp.stack([h_ref[0] * w0[:, None], h_ref[1] * w1[:, None]])
    den_ref[...] = jnp.stack([w0, w1])

    @pl.when(i == 0)
    def _():
        m16_ref[...] = jnp.stack([jnp.full((16,), m0), jnp.full((16,), m1)])


def _post2_body(acc_ref, den_ref, b2_ref, out_ref):
    o0 = acc_ref[0, :, :7] / den_ref[0][:, None]
    o1 = acc_ref[1, :, :7] / den_ref[1][:, None]
    z = 0.5 * (o0 + o1) + b2_ref[0, :]
    z = z - jnp.max(z, axis=-1, keepdims=True)
    out_ref[...] = z - jnp.log(jnp.sum(jnp.exp(z), axis=-1, keepdims=True))


def _make_sc_edge(n2, e_pad):
    rpt = n2 // _NS
    n_stage = 34
    rpt_h = rpt // n_stage
    assert rpt_h % 8 == 0 and rpt_h * n_stage == rpt
    ept = e_pad // _NS
    cpt = ept // _K
    c_dim = 16

    mesh = plsc.VectorSubcoreMesh(core_axis_name="c", subcore_axis_name="s")

    @functools.partial(
        pl.kernel,
        out_type=[
            jax.ShapeDtypeStruct((_NC * n2, c_dim), jnp.float32),
            jax.ShapeDtypeStruct((_NC * n2,), jnp.float32),
        ],
        mesh=mesh,
        compiler_params=pltpu.CompilerParams(use_tc_tiling_on_sc=False),
        scratch_types=[
            pltpu.VMEM((_K,), jnp.int32),
            pltpu.VMEM((_K,), jnp.int32),
            pltpu.VMEM((_K,), jnp.int32),
            pltpu.VMEM((_K,), jnp.float32),
            pltpu.VMEM((_K,), jnp.float32),
            pltpu.VMEM((_K,), jnp.float32),
            pltpu.VMEM((_K, c_dim), jnp.float32),
            pltpu.VMEM((_LANES,), jnp.float32),
            pltpu.VMEM((rpt_h, c_dim), jnp.float32),
            pltpu.VMEM((rpt_h,), jnp.float32),
            pltpu.VMEM_SHARED((n2, c_dim), jnp.float32),
            pltpu.VMEM_SHARED((n2,), jnp.float32),
            pltpu.VMEM_SHARED((n2,), jnp.float32),
            pltpu.VMEM_SHARED((n2,), jnp.float32),
            pltpu.SemaphoreType.DMA,
            pltpu.SemaphoreType.DMA,
            pltpu.SemaphoreType.DMA,
        ],
    )
    def sc_edge(src_h, dst_h, h_h, as_h, ad_h, acc0_h, den0_h, m_h,
                acc_o, den_o,
                idx_s, idx_d, idx_s2, asv, adv, wv, rows, mv,
                stage, vstage, acc_sh, den_sh, as_sh, ad_sh,
                sem0, sem1, sem2):
        c = lax.axis_index("c")
        s = lax.axis_index("s")
        hbase = c * n2
        rbase = s * rpt

        pltpu.sync_copy(m_h.at[pl.ds(c * _LANES, _LANES)], mv)

        @pl.loop(0, n_stage)
        def _init(j):
            off = rbase + j * rpt_h
            for vec_h, vec_sh in ((as_h, as_sh), (ad_h, ad_sh),
                                  (den0_h, den_sh)):
                pltpu.sync_copy(vec_h.at[pl.ds(hbase + off, rpt_h)], vstage)
                pltpu.sync_copy(vstage, vec_sh.at[pl.ds(off, rpt_h)])
            pltpu.sync_copy(acc0_h.at[pl.ds(hbase + off, rpt_h)], stage)
            pltpu.sync_copy(stage, acc_sh.at[pl.ds(off, rpt_h)])

        plsc.subcore_barrier()

        ebase = s * ept
        mvec = mv[...]

        @pl.loop(0, cpt)
        def _chunk(j):
            b = ebase + j * _K
            cp_s = pltpu.async_copy(src_h.at[pl.ds(b, _K)], idx_s, sem0)
            cp_d = pltpu.async_copy(dst_h.at[pl.ds(b, _K)], idx_d, sem1)
            cp_s.wait()
            cp_d.wait()
            for r in range(_K // _LANES):
                sl = pl.ds(r * _LANES, _LANES)
                idx_s2[sl] = idx_s[sl] + hbase
            g_as = pltpu.async_copy(as_sh.at[idx_s], asv, sem0)
            g_ad = pltpu.async_copy(ad_sh.at[idx_d], adv, sem1)
            g_h = pltpu.async_copy(h_h.at[idx_s2], rows, sem2)
            g_as.wait()
            g_ad.wait()
            for r in range(_K // _LANES):
                sl = pl.ds(r * _LANES, _LANES)
                e = asv[sl] + adv[sl]
                e = jnp.where(e > 0, e, 0.2 * e)
                wv[sl] = jnp.exp(e - mvec)
            pltpu.sync_copy(wv, den_sh.at[idx_d], add=True)
            g_h.wait()
            for r in range(_K // _LANES):
                wvec = wv[pl.ds(r * _LANES, _LANES)]
                for kk in range(_LANES):
                    k = r * _LANES + kk
                    rows[k, :] = rows[k, :] * wvec[kk]
            pltpu.sync_copy(rows, acc_sh.at[idx_d], add=True)

        plsc.subcore_barrier()

        @pl.loop(0, n_stage)
        def _readback(j):
            off = rbase + j * rpt_h
            pltpu.sync_copy(acc_sh.at[pl.ds(off, rpt_h)], stage)
            pltpu.sync_copy(stage, acc_o.at[pl.ds(hbase + off, rpt_h)])
            pltpu.sync_copy(den_sh.at[pl.ds(off, rpt_h)], vstage)
            pltpu.sync_copy(vstage, den_o.at[pl.ds(hbase + off, rpt_h)])

    return sc_edge


def kernel(x, edge_index, W1, a_src1, a_dst1, b1, W2, a_src2, a_dst2, b2):
    n = x.shape[0]
    e = edge_index.shape[1]
    n2 = ((n + 8 + 127) // 128) * 128
    e_pad = ((e + _NS * _K - 1) // (_NS * _K)) * (_NS * _K)
    grid_b = 4352 if n2 % 4352 == 0 else 128
    g = n2 // grid_b

    xp = jnp.pad(x, ((0, n2 - n), (0, 0)))
    src_p = jnp.concatenate(
        [edge_index[0], jnp.zeros((e_pad - e,), jnp.int32)])
    dst_p = jnp.concatenate(
        [edge_index[1], jnp.full((e_pad - e,), n, jnp.int32)])

    f32 = jnp.float32
    node_vec = jax.ShapeDtypeStruct((2, n2), f32)
    node_mat = jax.ShapeDtypeStruct((2, n2, 16), f32)

    pro1 = pl.pallas_call(
        _pro1_body,
        grid=(g,),
        in_specs=[
            pl.BlockSpec((grid_b, 3), lambda i: (i, 0)),
            pl.BlockSpec((3, 32), lambda i: (0, 0)),
            pl.BlockSpec((2, 16), lambda i: (0, 0)),
            pl.BlockSpec((2, 16), lambda i: (0, 0)),
        ],
        out_specs=[
            pl.BlockSpec((2, grid_b, 16), lambda i: (0, i, 0)),
            pl.BlockSpec((2, grid_b), lambda i: (0, i)),
            pl.BlockSpec((2, grid_b), lambda i: (0, i)),
            pl.BlockSpec((1, 8), lambda i: (0, 0)),
        ],
        out_shape=[node_mat, node_vec, node_vec,
                   jax.ShapeDtypeStruct((1, 8), f32)],
    )
    init_k = pl.pallas_call(
        _init_body,
        grid=(g,),
        in_specs=[
            pl.BlockSpec((2, grid_b, 16), lambda i: (0, i, 0)),
            pl.BlockSpec((2, grid_b), lambda i: (0, i)),
            pl.BlockSpec((2, grid_b), lambda i: (0, i)),
            pl.BlockSpec((1, 8), lambda i: (0, 0)),
        ],
        out_specs=[
            pl.BlockSpec((2, grid_b, 16), lambda i: (0, i, 0)),
            pl.BlockSpec((2, grid_b), lambda i: (0, i)),
            pl.BlockSpec((2, 16), lambda i: (0, 0)),
        ],
        out_shape=[node_mat, node_vec,
                   jax.ShapeDtypeStruct((2, 16), f32)],
    )
    mid = pl.pallas_call(
        _mid_body,
        grid=(g,),
        in_specs=[
            pl.BlockSpec((2, grid_b, 16), lambda i: (0, i, 0)),
            pl.BlockSpec((2, grid_b), lambda i: (0, i)),
            pl.BlockSpec((1, 32), lambda i: (0, 0)),
            pl.BlockSpec((32, 14), lambda i: (0, 0)),
            pl.BlockSpec((2, 7), lambda i: (0, 0)),
            pl.BlockSpec((2, 7), lambda i: (0, 0)),
        ],
        out_specs=[
            pl.BlockSpec((2, grid_b, 16), lambda i: (0, i, 0)),
            pl.BlockSpec((2, grid_b), lambda i: (0, i)),
            pl.BlockSpec((2, grid_b), lambda i: (0, i)),
            pl.BlockSpec((1, 8), lambda i: (0, 0)),
        ],
        out_shape=[node_mat, node_vec, node_vec,
                   jax.ShapeDtypeStruct((1, 8), f32)],
    )
    post2 = pl.pallas_call(
        _post2_body,
        grid=(g,),
        in_specs=[
            pl.BlockSpec((2, grid_b, 16), lambda i: (0, i, 0)),
            pl.BlockSpec((2, grid_b), lambda i: (0, i)),
            pl.BlockSpec((1, 7), lambda i: (0, 0)),
        ],
        out_specs=pl.BlockSpec((grid_b, 7), lambda i: (i, 0)),
        out_shape=jax.ShapeDtypeStruct((n2, 7), f32),
    )
    sc_edge = _make_sc_edge(n2, e_pad)

    def flat(a):
        return a.reshape((-1,) + a.shape[2:])

    h_st, as_st, ad_st, m1 = pro1(xp, W1, a_src1, a_dst1)
    acc0, den0, m16 = init_k(h_st, as_st, ad_st, m1)
    acc1, den1 = sc_edge(src_p, dst_p, flat(h_st), flat(as_st), flat(ad_st),
                         flat(acc0), flat(den0), m16.reshape(-1))
    acc1 = acc1.reshape(2, n2, 16)
    den1 = den1.reshape(2, n2)

    h2_st, as2, ad2, m2 = mid(acc1, den1, b1.reshape(1, 32), W2,
                              a_src2, a_dst2)
    acc02, den02, m216 = init_k(h2_st, as2, ad2, m2)
    acc2, den2 = sc_edge(src_p, dst_p, flat(h2_st), flat(as2), flat(ad2),
                         flat(acc02), flat(den02), m216.reshape(-1))
    acc2 = acc2.reshape(2, n2, 16)
    den2 = den2.reshape(2, n2)

    out = post2(acc2, den2, b2.reshape(1, 7))
    return out[:n]


# speedup vs baseline: 166.8018x; 1.4222x over previous
import functools

import jax
import jax.numpy as jnp
from jax import lax
from jax.experimental import pallas as pl
from jax.experimental.pallas import tpu as pltpu
from jax.experimental.pallas import tpu_sc as plsc

_NC = 2
_NS = 16
_LANES = 16
_K = 128


def _leaky(v):
    return jnp.where(v > 0, v, 0.2 * v)


def _pro1_body(x_ref, w_ref, s_ref, d_ref, h_ref, as_ref, ad_ref, m_ref):
    i = pl.program_id(0)
    x = x_ref[...]
    h0 = jnp.dot(x, w_ref[:, :16], preferred_element_type=jnp.float32)
    h1 = jnp.dot(x, w_ref[:, 16:], preferred_element_type=jnp.float32)
    as0 = jnp.sum(h0 * s_ref[0, :], axis=-1)
    as1 = jnp.sum(h1 * s_ref[1, :], axis=-1)
    ad0 = jnp.sum(h0 * d_ref[0, :], axis=-1)
    ad1 = jnp.sum(h1 * d_ref[1, :], axis=-1)
    h_ref[...] = jnp.stack([h0, h1])
    as_ref[...] = jnp.stack([as0, as1])
    ad_ref[...] = jnp.stack([ad0, ad1])
    part = jnp.stack([jnp.max(as0), jnp.max(as1), jnp.max(ad0), jnp.max(ad1)])
    part = jnp.concatenate([part, part]).reshape(1, 8)

    @pl.when(i == 0)
    def _():
        m_ref[...] = part

    @pl.when(i != 0)
    def _():
        m_ref[...] = jnp.maximum(m_ref[...], part)


def _mid_body(acc_ref, den_ref, b1_ref, w2_ref, s2_ref, d2_ref,
              h_ref, as_ref, ad_ref, m_ref):
    i = pl.program_id(0)
    b = acc_ref.shape[1]
    g0 = acc_ref[0] / den_ref[0][:, None]
    g1 = acc_ref[1] / den_ref[1][:, None]
    xh = jnp.concatenate([g0, g1], axis=-1) + b1_ref[0, :]
    xh = jnp.maximum(xh, 0.0)
    h2 = jnp.dot(xh, w2_ref[...], preferred_element_type=jnp.float32)
    h20 = h2[:, :7]
    h21 = h2[:, 7:14]
    as0 = jnp.sum(h20 * s2_ref[0, :], axis=-1)
    as1 = jnp.sum(h21 * s2_ref[1, :], axis=-1)
    ad0 = jnp.sum(h20 * d2_ref[0, :], axis=-1)
    ad1 = jnp.sum(h21 * d2_ref[1, :], axis=-1)
    z = jnp.zeros((b, 9), jnp.float32)
    h_ref[...] = jnp.stack([jnp.concatenate([h20, z], axis=-1),
                            jnp.concatenate([h21, z], axis=-1)])
    as_ref[...] = jnp.stack([as0, as1])
    ad_ref[...] = jnp.stack([ad0, ad1])
    part = jnp.stack([jnp.max(as0), jnp.max(as1), jnp.max(ad0), jnp.max(ad1)])
    part = jnp.concatenate([part, part]).reshape(1, 8)

    @pl.when(i == 0)
    def _():
        m_ref[...] = part

    @pl.when(i != 0)
    def _():
        m_ref[...] = jnp.maximum(m_ref[...], part)


def _init_body(h_ref, as_ref, ad_ref, m_ref, acc_ref, den_ref, m16_ref):
    i = pl.program_id(0)
    m0 = _leaky(m_ref[0, 0] + m_ref[0, 2])
    m1 = _leaky(m_ref[0, 1] + m_ref[0, 3])
    w0 = jnp.exp(_leaky(as_ref[0] + ad_ref[0]) - m0)
    w1 = jnp.exp(_leaky(as_ref[1] + ad_ref[1]) - m1)
    acc_ref[...] = jnp.stack([h_ref[0] * w0[:, None], h_ref[1] * w1[:, None]])
    den_ref[...] = jnp.stack([w0, w1])

    @pl.when(i == 0)
    def _():
        m16_ref[...] = jnp.stack([jnp.full((16,), m0), jnp.full((16,), m1)])


def _post2_body(acc_ref, den_ref, b2_ref, out_ref):
    o0 = acc_ref[0, :, :7] / den_ref[0][:, None]
    o1 = acc_ref[1, :, :7] / den_ref[1][:, None]
    z = 0.5 * (o0 + o1) + b2_ref[0, :]
    z = z - jnp.max(z, axis=-1, keepdims=True)
    out_ref[...] = z - jnp.log(jnp.sum(jnp.exp(z), axis=-1, keepdims=True))


def _make_sc_edge(n2, e_pad):
    rpt = n2 // _NS
    ept = e_pad // _NS
    cpt = ept // _K
    assert cpt % 2 == 0 and cpt >= 4
    c_dim = 16

    mesh = plsc.VectorSubcoreMesh(core_axis_name="c", subcore_axis_name="s")

    dbl = lambda t: [t, t]
    @functools.partial(
        pl.kernel,
        out_type=[
            jax.ShapeDtypeStruct((_NC * n2, c_dim), jnp.float32),
            jax.ShapeDtypeStruct((_NC * n2,), jnp.float32),
        ],
        mesh=mesh,
        compiler_params=pltpu.CompilerParams(use_tc_tiling_on_sc=False),
        scratch_types=(
            dbl(pltpu.VMEM((_K,), jnp.int32))
            + dbl(pltpu.VMEM((_K,), jnp.int32))
            + dbl(pltpu.VMEM((_K,), jnp.int32))
            + dbl(pltpu.VMEM((_K,), jnp.float32))
            + dbl(pltpu.VMEM((_K,), jnp.float32))
            + dbl(pltpu.VMEM((_K,), jnp.float32))
            + dbl(pltpu.VMEM((_K, c_dim), jnp.float32))
            + [pltpu.VMEM((_LANES,), jnp.float32)]
            + [pltpu.VMEM_SHARED((n2, c_dim), jnp.float32),
               pltpu.VMEM_SHARED((n2,), jnp.float32),
               pltpu.VMEM_SHARED((n2,), jnp.float32),
               pltpu.VMEM_SHARED((n2,), jnp.float32)]
            + [pltpu.SemaphoreType.DMA] * 10
        ),
    )
    def sc_edge(src_h, dst_h, h_h, as_h, ad_h, acc0_h, den0_h, m_h,
                acc_o, den_o,
                is0, is1, id0, id1, i20, i21, av0, av1, bv0, bv1,
                wv0, wv1, ro0, ro1, mv,
                acc_sh, den_sh, as_sh, ad_sh,
                ss0, ss1, sd0, sd1, sa0, sa1, sb0, sb1, sh0, sh1):
        c = lax.axis_index("c")
        s = lax.axis_index("s")
        hbase = c * n2
        rbase = s * rpt
        IS, ID, I2 = (is0, is1), (id0, id1), (i20, i21)
        AV, BV, WV, RO = (av0, av1), (bv0, bv1), (wv0, wv1), (ro0, ro1)
        SS, SD, SA, SB, SH = ((ss0, ss1), (sd0, sd1), (sa0, sa1),
                              (sb0, sb1), (sh0, sh1))

        pltpu.sync_copy(m_h.at[pl.ds(c * _LANES, _LANES)], mv)

        pltpu.sync_copy(as_h.at[pl.ds(hbase + rbase, rpt)],
                        as_sh.at[pl.ds(rbase, rpt)])
        pltpu.sync_copy(ad_h.at[pl.ds(hbase + rbase, rpt)],
                        ad_sh.at[pl.ds(rbase, rpt)])
        pltpu.sync_copy(den0_h.at[pl.ds(hbase + rbase, rpt)],
                        den_sh.at[pl.ds(rbase, rpt)])
        pltpu.sync_copy(acc0_h.at[pl.ds(hbase + rbase, rpt)],
                        acc_sh.at[pl.ds(rbase, rpt)])
        plsc.subcore_barrier()

        ebase = s * ept
        mvec = mv[...]

        def fire_idx(chunk, p):
            b = ebase + chunk * _K
            pltpu.async_copy(src_h.at[pl.ds(b, _K)], IS[p], SS[p])
            pltpu.async_copy(dst_h.at[pl.ds(b, _K)], ID[p], SD[p])

        def wait_idx(p):
            z = pl.ds(0, _K)
            pltpu.make_async_copy(src_h.at[z], IS[p], SS[p]).wait()
            pltpu.make_async_copy(dst_h.at[z], ID[p], SD[p]).wait()

        def fire_gathers(p):
            for r in range(_K // _LANES):
                sl = pl.ds(r * _LANES, _LANES)
                I2[p][sl] = IS[p][sl] + hbase
            pltpu.async_copy(as_sh.at[IS[p]], AV[p], SA[p])
            pltpu.async_copy(ad_sh.at[ID[p]], BV[p], SB[p])
            pltpu.async_copy(h_h.at[I2[p]], RO[p], SH[p])

        def process(p):
            pltpu.make_async_copy(as_sh.at[IS[p]], AV[p], SA[p]).wait()
            pltpu.make_async_copy(ad_sh.at[ID[p]], BV[p], SB[p]).wait()
            for r in range(_K // _LANES):
                sl = pl.ds(r * _LANES, _LANES)
                e = AV[p][sl] + BV[p][sl]
                e = jnp.where(e > 0, e, 0.2 * e)
                WV[p][sl] = jnp.exp(e - mvec)
            pltpu.sync_copy(WV[p], den_sh.at[ID[p]], add=True)
            pltpu.make_async_copy(h_h.at[I2[p]], RO[p], SH[p]).wait()
            for r in range(_K // _LANES):
                wvec = WV[p][pl.ds(r * _LANES, _LANES)]
                for kk in range(_LANES):
                    k = r * _LANES + kk
                    RO[p][k, :] = RO[p][k, :] * wvec[kk]
            pltpu.sync_copy(RO[p], acc_sh.at[ID[p]], add=True)

        def slot(chunk, p):
            @pl.when(chunk + 1 < cpt)
            def _():
                wait_idx(1 - p)
                fire_gathers(1 - p)
            process(p)
            @pl.when(chunk + 2 < cpt)
            def _():
                fire_idx(chunk + 2, p)

        fire_idx(0, 0)
        wait_idx(0)
        fire_gathers(0)
        fire_idx(1, 1)

        @pl.loop(0, cpt // 2)
        def _pair(t):
            slot(2 * t, 0)
            slot(2 * t + 1, 1)

        plsc.subcore_barrier()

        pltpu.sync_copy(acc_sh.at[pl.ds(rbase, rpt)],
                        acc_o.at[pl.ds(hbase + rbase, rpt)])
        pltpu.sync_copy(den_sh.at[pl.ds(rbase, rpt)],
                        den_o.at[pl.ds(hbase + rbase, rpt)])

    return sc_edge


def kernel(x, edge_index, W1, a_src1, a_dst1, b1, W2, a_src2, a_dst2, b2):
    n = x.shape[0]
    e = edge_index.shape[1]
    n2 = ((n + 8 + 127) // 128) * 128
    e_pad = ((e + _NS * _K - 1) // (_NS * _K)) * (_NS * _K)
    grid_b = 4352 if n2 % 4352 == 0 else 128
    g = n2 // grid_b

    xp = jnp.pad(x, ((0, n2 - n), (0, 0)))
    src_p = jnp.concatenate(
        [edge_index[0], jnp.zeros((e_pad - e,), jnp.int32)])
    dst_p = jnp.concatenate(
        [edge_index[1], jnp.full((e_pad - e,), n, jnp.int32)])

    f32 = jnp.float32
    node_vec = jax.ShapeDtypeStruct((2, n2), f32)
    node_mat = jax.ShapeDtypeStruct((2, n2, 16), f32)

    pro1 = pl.pallas_call(
        _pro1_body,
        grid=(g,),
        in_specs=[
            pl.BlockSpec((grid_b, 3), lambda i: (i, 0)),
            pl.BlockSpec((3, 32), lambda i: (0, 0)),
            pl.BlockSpec((2, 16), lambda i: (0, 0)),
            pl.BlockSpec((2, 16), lambda i: (0, 0)),
        ],
        out_specs=[
            pl.BlockSpec((2, grid_b, 16), lambda i: (0, i, 0)),
            pl.BlockSpec((2, grid_b), lambda i: (0, i)),
            pl.BlockSpec((2, grid_b), lambda i: (0, i)),
            pl.BlockSpec((1, 8), lambda i: (0, 0)),
        ],
        out_shape=[node_mat, node_vec, node_vec,
                   jax.ShapeDtypeStruct((1, 8), f32)],
    )
    init_k = pl.pallas_call(
        _init_body,
        grid=(g,),
        in_specs=[
            pl.BlockSpec((2, grid_b, 16), lambda i: (0, i, 0)),
            pl.BlockSpec((2, grid_b), lambda i: (0, i)),
            pl.BlockSpec((2, grid_b), lambda i: (0, i)),
            pl.BlockSpec((1, 8), lambda i: (0, 0)),
        ],
        out_specs=[
            pl.BlockSpec((2, grid_b, 16), lambda i: (0, i, 0)),
            pl.BlockSpec((2, grid_b), lambda i: (0, i)),
            pl.BlockSpec((2, 16), lambda i: (0, 0)),
        ],
        out_shape=[node_mat, node_vec,
                   jax.ShapeDtypeStruct((2, 16), f32)],
    )
    mid = pl.pallas_call(
        _mid_body,
        grid=(g,),
        in_specs=[
            pl.BlockSpec((2, grid_b, 16), lambda i: (0, i, 0)),
            pl.BlockSpec((2, grid_b), lambda i: (0, i)),
            pl.BlockSpec((1, 32), lambda i: (0, 0)),
            pl.BlockSpec((32, 14), lambda i: (0, 0)),
            pl.BlockSpec((2, 7), lambda i: (0, 0)),
            pl.BlockSpec((2, 7), lambda i: (0, 0)),
        ],
        out_specs=[
            pl.BlockSpec((2, grid_b, 16), lambda i: (0, i, 0)),
            pl.BlockSpec((2, grid_b), lambda i: (0, i)),
            pl.BlockSpec((2, grid_b), lambda i: (0, i)),
            pl.BlockSpec((1, 8), lambda i: (0, 0)),
        ],
        out_shape=[node_mat, node_vec, node_vec,
                   jax.ShapeDtypeStruct((1, 8), f32)],
    )
    post2 = pl.pallas_call(
        _post2_body,
        grid=(g,),
        in_specs=[
            pl.BlockSpec((2, grid_b, 16), lambda i: (0, i, 0)),
            pl.BlockSpec((2, grid_b), lambda i: (0, i)),
            pl.BlockSpec((1, 7), lambda i: (0, 0)),
        ],
        out_specs=pl.BlockSpec((grid_b, 7), lambda i: (i, 0)),
        out_shape=jax.ShapeDtypeStruct((n2, 7), f32),
    )
    sc_edge = _make_sc_edge(n2, e_pad)

    def flat(a):
        return a.reshape((-1,) + a.shape[2:])

    h_st, as_st, ad_st, m1 = pro1(xp, W1, a_src1, a_dst1)
    acc0, den0, m16 = init_k(h_st, as_st, ad_st, m1)
    acc1, den1 = sc_edge(src_p, dst_p, flat(h_st), flat(as_st), flat(ad_st),
                         flat(acc0), flat(den0), m16.reshape(-1))
    acc1 = acc1.reshape(2, n2, 16)
    den1 = den1.reshape(2, n2)

    h2_st, as2, ad2, m2 = mid(acc1, den1, b1.reshape(1, 32), W2,
                              a_src2, a_dst2)
    acc02, den02, m216 = init_k(h2_st, as2, ad2, m2)
    acc2, den2 = sc_edge(src_p, dst_p, flat(h2_st), flat(as2), flat(ad2),
                         flat(acc02), flat(den02), m216.reshape(-1))
    acc2 = acc2.reshape(2, n2, 16)
    den2 = den2.reshape(2, n2)

    out = post2(acc2, den2, b2.reshape(1, 7))
    return out[:n]


# speedup vs baseline: 193.2062x; 1.1583x over previous
import functools

import jax
import jax.numpy as jnp
from jax import lax
from jax.experimental import pallas as pl
from jax.experimental.pallas import tpu as pltpu
from jax.experimental.pallas import tpu_sc as plsc

_NC = 2
_NS = 16
_LANES = 16
_K = 128


def _leaky(v):
    return jnp.where(v > 0, v, 0.2 * v)


def _pro1_body(x_ref, w_ref, s_ref, d_ref, h_ref, as_ref, ad_ref, m_ref):
    i = pl.program_id(0)
    x = x_ref[...]
    h0 = jnp.dot(x, w_ref[:, :16], preferred_element_type=jnp.float32)
    h1 = jnp.dot(x, w_ref[:, 16:], preferred_element_type=jnp.float32)
    as0 = jnp.sum(h0 * s_ref[0, :], axis=-1)
    as1 = jnp.sum(h1 * s_ref[1, :], axis=-1)
    ad0 = jnp.sum(h0 * d_ref[0, :], axis=-1)
    ad1 = jnp.sum(h1 * d_ref[1, :], axis=-1)
    h_ref[...] = jnp.stack([h0, h1])
    as_ref[...] = jnp.stack([as0, as1])
    ad_ref[...] = jnp.stack([ad0, ad1])
    part = jnp.stack([jnp.max(as0), jnp.max(as1), jnp.max(ad0), jnp.max(ad1)])
    part = jnp.concatenate([part, part]).reshape(1, 8)

    @pl.when(i == 0)
    def _():
        m_ref[...] = part

    @pl.when(i != 0)
    def _():
        m_ref[...] = jnp.maximum(m_ref[...], part)


def _mid_body(acc_ref, den_ref, b1_ref, w2_ref, s2_ref, d2_ref,
              h_ref, as_ref, ad_ref, m_ref):
    i = pl.program_id(0)
    b = acc_ref.shape[1]
    g0 = acc_ref[0] / den_ref[0][:, None]
    g1 = acc_ref[1] / den_ref[1][:, None]
    xh = jnp.concatenate([g0, g1], axis=-1) + b1_ref[0, :]
    xh = jnp.maximum(xh, 0.0)
    h2 = jnp.dot(xh, w2_ref[...], preferred_element_type=jnp.float32)
    h20 = h2[:, :7]
    h21 = h2[:, 7:14]
    as0 = jnp.sum(h20 * s2_ref[0, :], axis=-1)
    as1 = jnp.sum(h21 * s2_ref[1, :], axis=-1)
    ad0 = jnp.sum(h20 * d2_ref[0, :], axis=-1)
    ad1 = jnp.sum(h21 * d2_ref[1, :], axis=-1)
    z = jnp.zeros((b, 9), jnp.float32)
    h_ref[...] = jnp.stack([jnp.concatenate([h20, z], axis=-1),
                            jnp.concatenate([h21, z], axis=-1)])
    as_ref[...] = jnp.stack([as0, as1])
    ad_ref[...] = jnp.stack([ad0, ad1])
    part = jnp.stack([jnp.max(as0), jnp.max(as1), jnp.max(ad0), jnp.max(ad1)])
    part = jnp.concatenate([part, part]).reshape(1, 8)

    @pl.when(i == 0)
    def _():
        m_ref[...] = part

    @pl.when(i != 0)
    def _():
        m_ref[...] = jnp.maximum(m_ref[...], part)


def _init_body(h_ref, as_ref, ad_ref, m_ref, acc_ref, den_ref, m16_ref):
    i = pl.program_id(0)
    m0 = _leaky(m_ref[0, 0] + m_ref[0, 2])
    m1 = _leaky(m_ref[0, 1] + m_ref[0, 3])
    w0 = jnp.exp(_leaky(as_ref[0] + ad_ref[0]) - m0)
    w1 = jnp.exp(_leaky(as_ref[1] + ad_ref[1]) - m1)
    acc_ref[...] = jnp.stack([h_ref[0] * w0[:, None], h_ref[1] * w1[:, None]])
    den_ref[...] = jnp.stack([w0, w1])

    @pl.when(i == 0)
    def _():
        m16_ref[...] = jnp.stack([jnp.full((16,), m0), jnp.full((16,), m1)])


def _post2_body(acc_ref, den_ref, b2_ref, out_ref):
    o0 = acc_ref[0, :, :7] / den_ref[0][:, None]
    o1 = acc_ref[1, :, :7] / den_ref[1][:, None]
    z = 0.5 * (o0 + o1) + b2_ref[0, :]
    z = z - jnp.max(z, axis=-1, keepdims=True)
    out_ref[...] = z - jnp.log(jnp.sum(jnp.exp(z), axis=-1, keepdims=True))


def _make_sc_edge(n2, e_pad):
    rpt = n2 // _NS
    ept = e_pad // _NS
    cpt = ept // _K
    assert cpt % 2 == 0 and cpt >= 4
    c_dim = 16

    mesh = plsc.VectorSubcoreMesh(core_axis_name="c", subcore_axis_name="s")

    dbl = lambda t: [t, t]
    @functools.partial(
        pl.kernel,
        out_type=[
            jax.ShapeDtypeStruct((_NC * n2, c_dim), jnp.float32),
            jax.ShapeDtypeStruct((_NC * n2,), jnp.float32),
        ],
        mesh=mesh,
        compiler_params=pltpu.CompilerParams(use_tc_tiling_on_sc=False),
        scratch_types=(
            dbl(pltpu.VMEM((_K,), jnp.int32))
            + dbl(pltpu.VMEM((_K,), jnp.int32))
            + dbl(pltpu.VMEM((_K,), jnp.int32))
            + dbl(pltpu.VMEM((_K,), jnp.float32))
            + dbl(pltpu.VMEM((_K,), jnp.float32))
            + dbl(pltpu.VMEM((_K,), jnp.float32))
            + dbl(pltpu.VMEM((_K, c_dim), jnp.float32))
            + dbl(pltpu.VMEM((_K,), jnp.int32))
            + [pltpu.VMEM((_LANES,), jnp.float32)]
            + [pltpu.VMEM_SHARED((n2, c_dim), jnp.float32),
               pltpu.VMEM_SHARED((n2,), jnp.float32),
               pltpu.VMEM_SHARED((n2,), jnp.float32),
               pltpu.VMEM_SHARED((n2,), jnp.float32)]
            + [pltpu.SemaphoreType.DMA] * 14
        ),
    )
    def sc_edge(src_h, dst_h, h_h, as_h, ad_h, acc0_h, den0_h, m_h,
                acc_o, den_o,
                is0, is1, id0, id1, i20, i21, av0, av1, bv0, bv1,
                wv0, wv1, ro0, ro1, dd0, dd1, mv,
                acc_sh, den_sh, as_sh, ad_sh,
                ss0, ss1, sd0, sd1, sa0, sa1, sb0, sb1, sh0, sh1,
                sn0, sn1, sc0, sc1):
        c = lax.axis_index("c")
        s = lax.axis_index("s")
        hbase = c * n2
        rbase = s * rpt
        IS, ID, I2 = (is0, is1), (id0, id1), (i20, i21)
        AV, BV, WV, RO = (av0, av1), (bv0, bv1), (wv0, wv1), (ro0, ro1)
        DD = (dd0, dd1)
        SS, SD, SA, SB, SH = ((ss0, ss1), (sd0, sd1), (sa0, sa1),
                              (sb0, sb1), (sh0, sh1))
        SN, SC = (sn0, sn1), (sc0, sc1)

        pltpu.sync_copy(m_h.at[pl.ds(c * _LANES, _LANES)], mv)

        pltpu.sync_copy(as_h.at[pl.ds(hbase + rbase, rpt)],
                        as_sh.at[pl.ds(rbase, rpt)])
        pltpu.sync_copy(ad_h.at[pl.ds(hbase + rbase, rpt)],
                        ad_sh.at[pl.ds(rbase, rpt)])
        pltpu.sync_copy(den0_h.at[pl.ds(hbase + rbase, rpt)],
                        den_sh.at[pl.ds(rbase, rpt)])
        pltpu.sync_copy(acc0_h.at[pl.ds(hbase + rbase, rpt)],
                        acc_sh.at[pl.ds(rbase, rpt)])
        plsc.subcore_barrier()

        ebase = s * ept
        mvec = mv[...]

        def fire_idx(chunk, p):
            b = ebase + chunk * _K
            pltpu.async_copy(src_h.at[pl.ds(b, _K)], IS[p], SS[p])
            pltpu.async_copy(dst_h.at[pl.ds(b, _K)], ID[p], SD[p])

        def wait_idx(p):
            z = pl.ds(0, _K)
            pltpu.make_async_copy(src_h.at[z], IS[p], SS[p]).wait()
            pltpu.make_async_copy(dst_h.at[z], ID[p], SD[p]).wait()

        def fire_gathers(p):
            for r in range(_K // _LANES):
                sl = pl.ds(r * _LANES, _LANES)
                I2[p][sl] = IS[p][sl] + hbase
            pltpu.async_copy(as_sh.at[IS[p]], AV[p], SA[p])
            pltpu.async_copy(ad_sh.at[ID[p]], BV[p], SB[p])
            pltpu.async_copy(h_h.at[I2[p]], RO[p], SH[p])

        def wait_den(p):
            pltpu.make_async_copy(WV[p], den_sh.at[DD[p]], SN[p]).wait()

        def wait_acc(p):
            pltpu.make_async_copy(RO[p], acc_sh.at[DD[p]], SC[p]).wait()

        def process(chunk, p):
            pltpu.make_async_copy(as_sh.at[IS[p]], AV[p], SA[p]).wait()
            pltpu.make_async_copy(ad_sh.at[ID[p]], BV[p], SB[p]).wait()
            @pl.when(chunk >= 2)
            def _():
                wait_den(p)
            for r in range(_K // _LANES):
                sl = pl.ds(r * _LANES, _LANES)
                e = AV[p][sl] + BV[p][sl]
                e = jnp.where(e > 0, e, 0.2 * e)
                WV[p][sl] = jnp.exp(e - mvec)
            for r in range(_K // _LANES):
                sl = pl.ds(r * _LANES, _LANES)
                DD[p][sl] = ID[p][sl]
            pltpu.async_copy(WV[p], den_sh.at[DD[p]], SN[p], add=True)
            pltpu.make_async_copy(h_h.at[I2[p]], RO[p], SH[p]).wait()
            for r in range(_K // _LANES):
                wvec = WV[p][pl.ds(r * _LANES, _LANES)]
                for kk in range(_LANES):
                    k = r * _LANES + kk
                    RO[p][k, :] = RO[p][k, :] * wvec[kk]
            pltpu.async_copy(RO[p], acc_sh.at[DD[p]], SC[p], add=True)

        def slot(chunk, p):
            @pl.when(chunk + 1 < cpt)
            def _():
                wait_idx(1 - p)
                @pl.when(chunk >= 1)
                def _():
                    wait_acc(1 - p)
                fire_gathers(1 - p)
            process(chunk, p)
            @pl.when(chunk + 2 < cpt)
            def _():
                fire_idx(chunk + 2, p)

        fire_idx(0, 0)
        wait_idx(0)
        fire_gathers(0)
        fire_idx(1, 1)

        @pl.loop(0, cpt // 2)
        def _pair(t):
            slot(2 * t, 0)
            slot(2 * t + 1, 1)

        wait_den(0)
        wait_den(1)
        wait_acc(0)
        wait_acc(1)
        plsc.subcore_barrier()

        pltpu.sync_copy(acc_sh.at[pl.ds(rbase, rpt)],
                        acc_o.at[pl.ds(hbase + rbase, rpt)])
        pltpu.sync_copy(den_sh.at[pl.ds(rbase, rpt)],
                        den_o.at[pl.ds(hbase + rbase, rpt)])

    return sc_edge


def kernel(x, edge_index, W1, a_src1, a_dst1, b1, W2, a_src2, a_dst2, b2):
    n = x.shape[0]
    e = edge_index.shape[1]
    n2 = ((n + 8 + 127) // 128) * 128
    e_pad = ((e + _NS * _K - 1) // (_NS * _K)) * (_NS * _K)
    grid_b = 4352 if n2 % 4352 == 0 else 128
    g = n2 // grid_b

    xp = jnp.pad(x, ((0, n2 - n), (0, 0)))
    src_p = jnp.concatenate(
        [edge_index[0], jnp.zeros((e_pad - e,), jnp.int32)])
    dst_p = jnp.concatenate(
        [edge_index[1], jnp.full((e_pad - e,), n, jnp.int32)])

    f32 = jnp.float32
    node_vec = jax.ShapeDtypeStruct((2, n2), f32)
    node_mat = jax.ShapeDtypeStruct((2, n2, 16), f32)

    pro1 = pl.pallas_call(
        _pro1_body,
        grid=(g,),
        in_specs=[
            pl.BlockSpec((grid_b, 3), lambda i: (i, 0)),
            pl.BlockSpec((3, 32), lambda i: (0, 0)),
            pl.BlockSpec((2, 16), lambda i: (0, 0)),
            pl.BlockSpec((2, 16), lambda i: (0, 0)),
        ],
        out_specs=[
            pl.BlockSpec((2, grid_b, 16), lambda i: (0, i, 0)),
            pl.BlockSpec((2, grid_b), lambda i: (0, i)),
            pl.BlockSpec((2, grid_b), lambda i: (0, i)),
            pl.BlockSpec((1, 8), lambda i: (0, 0)),
        ],
        out_shape=[node_mat, node_vec, node_vec,
                   jax.ShapeDtypeStruct((1, 8), f32)],
    )
    init_k = pl.pallas_call(
        _init_body,
        grid=(g,),
        in_specs=[
            pl.BlockSpec((2, grid_b, 16), lambda i: (0, i, 0)),
            pl.BlockSpec((2, grid_b), lambda i: (0, i)),
            pl.BlockSpec((2, grid_b), lambda i: (0, i)),
            pl.BlockSpec((1, 8), lambda i: (0, 0)),
        ],
        out_specs=[
            pl.BlockSpec((2, grid_b, 16), lambda i: (0, i, 0)),
            pl.BlockSpec((2, grid_b), lambda i: (0, i)),
            pl.BlockSpec((2, 16), lambda i: (0, 0)),
        ],
        out_shape=[node_mat, node_vec,
                   jax.ShapeDtypeStruct((2, 16), f32)],
    )
    mid = pl.pallas_call(
        _mid_body,
        grid=(g,),
        in_specs=[
            pl.BlockSpec((2, grid_b, 16), lambda i: (0, i, 0)),
            pl.BlockSpec((2, grid_b), lambda i: (0, i)),
            pl.BlockSpec((1, 32), lambda i: (0, 0)),
            pl.BlockSpec((32, 14), lambda i: (0, 0)),
            pl.BlockSpec((2, 7), lambda i: (0, 0)),
            pl.BlockSpec((2, 7), lambda i: (0, 0)),
        ],
        out_specs=[
            pl.BlockSpec((2, grid_b, 16), lambda i: (0, i, 0)),
            pl.BlockSpec((2, grid_b), lambda i: (0, i)),
            pl.BlockSpec((2, grid_b), lambda i: (0, i)),
            pl.BlockSpec((1, 8), lambda i: (0, 0)),
        ],
        out_shape=[node_mat, node_vec, node_vec,
                   jax.ShapeDtypeStruct((1, 8), f32)],
    )
    post2 = pl.pallas_call(
        _post2_body,
        grid=(g,),
        in_specs=[
            pl.BlockSpec((2, grid_b, 16), lambda i: (0, i, 0)),
            pl.BlockSpec((2, grid_b), lambda i: (0, i)),
            pl.BlockSpec((1, 7), lambda i: (0, 0)),
        ],
        out_specs=pl.BlockSpec((grid_b, 7), lambda i: (i, 0)),
        out_shape=jax.ShapeDtypeStruct((n2, 7), f32),
    )
    sc_edge = _make_sc_edge(n2, e_pad)

    def flat(a):
        return a.reshape((-1,) + a.shape[2:])

    h_st, as_st, ad_st, m1 = pro1(xp, W1, a_src1, a_dst1)
    acc0, den0, m16 = init_k(h_st, as_st, ad_st, m1)
    acc1, den1 = sc_edge(src_p, dst_p, flat(h_st), flat(as_st),
                         flat(ad_st), flat(acc0), flat(den0),
                         m16.reshape(-1))
    acc1 = acc1.reshape(2, n2, 16)
    den1 = den1.reshape(2, n2)

    h2_st, as2, ad2, m2 = mid(acc1, den1, b1.reshape(1, 32), W2,
                              a_src2, a_dst2)
    acc02, den02, m216 = init_k(h2_st, as2, ad2, m2)
    acc2, den2 = sc_edge(src_p, dst_p, flat(h2_st), flat(as2), flat(ad2),
                         flat(acc02), flat(den02), m216.reshape(-1))
    acc2 = acc2.reshape(2, n2, 16)
    den2 = den2.reshape(2, n2)

    out = post2(acc2, den2, b2.reshape(1, 7))
    return out[:n]


# speedup vs baseline: 237.0824x; 1.2271x over previous
import functools

import jax
import jax.numpy as jnp
from jax import lax
from jax.experimental import pallas as pl
from jax.experimental.pallas import tpu as pltpu
from jax.experimental.pallas import tpu_sc as plsc

_NC = 2
_NS = 16
_LANES = 16
_K = 256


def _leaky(v):
    return jnp.where(v > 0, v, 0.2 * v)


def _pro1_body(x_ref, w_ref, s_ref, d_ref, h_ref, as_ref, ad_ref, m_ref):
    i = pl.program_id(0)
    x = x_ref[...]
    h0 = jnp.dot(x, w_ref[:, :16], preferred_element_type=jnp.float32)
    h1 = jnp.dot(x, w_ref[:, 16:], preferred_element_type=jnp.float32)
    as0 = jnp.sum(h0 * s_ref[0, :], axis=-1)
    as1 = jnp.sum(h1 * s_ref[1, :], axis=-1)
    ad0 = jnp.sum(h0 * d_ref[0, :], axis=-1)
    ad1 = jnp.sum(h1 * d_ref[1, :], axis=-1)
    h_ref[...] = jnp.stack([h0, h1])
    as_ref[...] = jnp.stack([as0, as1])
    ad_ref[...] = jnp.stack([ad0, ad1])
    part = jnp.stack([jnp.max(as0), jnp.max(as1), jnp.max(ad0), jnp.max(ad1)])
    part = jnp.concatenate([part, part]).reshape(1, 8)

    @pl.when(i == 0)
    def _():
        m_ref[...] = part

    @pl.when(i != 0)
    def _():
        m_ref[...] = jnp.maximum(m_ref[...], part)


def _mid_body(acc_ref, den_ref, b1_ref, w2_ref, s2_ref, d2_ref,
              h_ref, as_ref, ad_ref, m_ref):
    i = pl.program_id(0)
    b = acc_ref.shape[1]
    g0 = acc_ref[0] / den_ref[0][:, None]
    g1 = acc_ref[1] / den_ref[1][:, None]
    xh = jnp.concatenate([g0, g1], axis=-1) + b1_ref[0, :]
    xh = jnp.maximum(xh, 0.0)
    h2 = jnp.dot(xh, w2_ref[...], preferred_element_type=jnp.float32)
    h20 = h2[:, :7]
    h21 = h2[:, 7:14]
    as0 = jnp.sum(h20 * s2_ref[0, :], axis=-1)
    as1 = jnp.sum(h21 * s2_ref[1, :], axis=-1)
    ad0 = jnp.sum(h20 * d2_ref[0, :], axis=-1)
    ad1 = jnp.sum(h21 * d2_ref[1, :], axis=-1)
    z = jnp.zeros((b, 9), jnp.float32)
    h_ref[...] = jnp.stack([jnp.concatenate([h20, z], axis=-1),
                            jnp.concatenate([h21, z], axis=-1)])
    as_ref[...] = jnp.stack([as0, as1])
    ad_ref[...] = jnp.stack([ad0, ad1])
    part = jnp.stack([jnp.max(as0), jnp.max(as1), jnp.max(ad0), jnp.max(ad1)])
    part = jnp.concatenate([part, part]).reshape(1, 8)

    @pl.when(i == 0)
    def _():
        m_ref[...] = part

    @pl.when(i != 0)
    def _():
        m_ref[...] = jnp.maximum(m_ref[...], part)


def _init_body(h_ref, as_ref, ad_ref, m_ref, acc_ref, den_ref, m16_ref):
    i = pl.program_id(0)
    m0 = _leaky(m_ref[0, 0] + m_ref[0, 2])
    m1 = _leaky(m_ref[0, 1] + m_ref[0, 3])
    w0 = jnp.exp(_leaky(as_ref[0] + ad_ref[0]) - m0)
    w1 = jnp.exp(_leaky(as_ref[1] + ad_ref[1]) - m1)
    acc_ref[...] = jnp.stack([h_ref[0] * w0[:, None], h_ref[1] * w1[:, None]])
    den_ref[...] = jnp.stack([w0, w1])

    @pl.when(i == 0)
    def _():
        m16_ref[...] = jnp.stack([jnp.full((16,), m0), jnp.full((16,), m1)])


def _post2_body(acc_ref, den_ref, b2_ref, out_ref):
    o0 = acc_ref[0, :, :7] / den_ref[0][:, None]
    o1 = acc_ref[1, :, :7] / den_ref[1][:, None]
    z = 0.5 * (o0 + o1) + b2_ref[0, :]
    z = z - jnp.max(z, axis=-1, keepdims=True)
    out_ref[...] = z - jnp.log(jnp.sum(jnp.exp(z), axis=-1, keepdims=True))


def _make_sc_edge(n2, e_pad):
    rpt = n2 // _NS
    ept = e_pad // _NS
    cpt = ept // _K
    assert cpt % 2 == 0 and cpt >= 4
    c_dim = 16

    mesh = plsc.VectorSubcoreMesh(core_axis_name="c", subcore_axis_name="s")

    dbl = lambda t: [t, t]
    @functools.partial(
        pl.kernel,
        out_type=[
            jax.ShapeDtypeStruct((_NC * n2, c_dim), jnp.float32),
            jax.ShapeDtypeStruct((_NC * n2,), jnp.float32),
        ],
        mesh=mesh,
        compiler_params=pltpu.CompilerParams(use_tc_tiling_on_sc=False),
        scratch_types=(
            dbl(pltpu.VMEM((_K,), jnp.int32))
            + dbl(pltpu.VMEM((_K,), jnp.int32))
            + dbl(pltpu.VMEM((_K,), jnp.int32))
            + dbl(pltpu.VMEM((_K,), jnp.float32))
            + dbl(pltpu.VMEM((_K,), jnp.float32))
            + dbl(pltpu.VMEM((_K,), jnp.float32))
            + dbl(pltpu.VMEM((_K, c_dim), jnp.float32))
            + dbl(pltpu.VMEM((_K,), jnp.int32))
            + [pltpu.VMEM((_LANES,), jnp.float32)]
            + [pltpu.VMEM_SHARED((n2, c_dim), jnp.float32),
               pltpu.VMEM_SHARED((n2,), jnp.float32),
               pltpu.VMEM_SHARED((n2,), jnp.float32),
               pltpu.VMEM_SHARED((n2,), jnp.float32)]
            + [pltpu.SemaphoreType.DMA] * 14
        ),
    )
    def sc_edge(src_h, dst_h, h_h, as_h, ad_h, acc0_h, den0_h, m_h,
                acc_o, den_o,
                is0, is1, id0, id1, i20, i21, av0, av1, bv0, bv1,
                wv0, wv1, ro0, ro1, dd0, dd1, mv,
                acc_sh, den_sh, as_sh, ad_sh,
                ss0, ss1, sd0, sd1, sa0, sa1, sb0, sb1, sh0, sh1,
                sn0, sn1, sc0, sc1):
        c = lax.axis_index("c")
        s = lax.axis_index("s")
        hbase = c * n2
        rbase = s * rpt
        IS, ID, I2 = (is0, is1), (id0, id1), (i20, i21)
        AV, BV, WV, RO = (av0, av1), (bv0, bv1), (wv0, wv1), (ro0, ro1)
        DD = (dd0, dd1)
        SS, SD, SA, SB, SH = ((ss0, ss1), (sd0, sd1), (sa0, sa1),
                              (sb0, sb1), (sh0, sh1))
        SN, SC = (sn0, sn1), (sc0, sc1)

        pltpu.sync_copy(m_h.at[pl.ds(c * _LANES, _LANES)], mv)

        pltpu.sync_copy(as_h.at[pl.ds(hbase + rbase, rpt)],
                        as_sh.at[pl.ds(rbase, rpt)])
        pltpu.sync_copy(ad_h.at[pl.ds(hbase + rbase, rpt)],
                        ad_sh.at[pl.ds(rbase, rpt)])
        pltpu.sync_copy(den0_h.at[pl.ds(hbase + rbase, rpt)],
                        den_sh.at[pl.ds(rbase, rpt)])
        pltpu.sync_copy(acc0_h.at[pl.ds(hbase + rbase, rpt)],
                        acc_sh.at[pl.ds(rbase, rpt)])
        plsc.subcore_barrier()

        ebase = s * ept
        mvec = mv[...]

        def fire_idx(chunk, p):
            b = ebase + chunk * _K
            pltpu.async_copy(src_h.at[pl.ds(b, _K)], IS[p], SS[p])
            pltpu.async_copy(dst_h.at[pl.ds(b, _K)], ID[p], SD[p])

        def wait_idx(p):
            z = pl.ds(0, _K)
            pltpu.make_async_copy(src_h.at[z], IS[p], SS[p]).wait()
            pltpu.make_async_copy(dst_h.at[z], ID[p], SD[p]).wait()

        def fire_gathers(p):
            for r in range(_K // _LANES):
                sl = pl.ds(r * _LANES, _LANES)
                I2[p][sl] = IS[p][sl] + hbase
            pltpu.async_copy(as_sh.at[IS[p]], AV[p], SA[p])
            pltpu.async_copy(ad_sh.at[ID[p]], BV[p], SB[p])
            pltpu.async_copy(h_h.at[I2[p]], RO[p], SH[p])

        def wait_den(p):
            pltpu.make_async_copy(WV[p], den_sh.at[DD[p]], SN[p]).wait()

        def wait_acc(p):
            pltpu.make_async_copy(RO[p], acc_sh.at[DD[p]], SC[p]).wait()

        def process(chunk, p):
            pltpu.make_async_copy(as_sh.at[IS[p]], AV[p], SA[p]).wait()
            pltpu.make_async_copy(ad_sh.at[ID[p]], BV[p], SB[p]).wait()
            @pl.when(chunk >= 2)
            def _():
                wait_den(p)
            for r in range(_K // _LANES):
                sl = pl.ds(r * _LANES, _LANES)
                e = AV[p][sl] + BV[p][sl]
                e = jnp.where(e > 0, e, 0.2 * e)
                WV[p][sl] = jnp.exp(e - mvec)
            for r in range(_K // _LANES):
                sl = pl.ds(r * _LANES, _LANES)
                DD[p][sl] = ID[p][sl]
            pltpu.async_copy(WV[p], den_sh.at[DD[p]], SN[p], add=True)
            pltpu.make_async_copy(h_h.at[I2[p]], RO[p], SH[p]).wait()
            for r in range(_K // _LANES):
                wvec = WV[p][pl.ds(r * _LANES, _LANES)]
                for kk in range(_LANES):
                    k = r * _LANES + kk
                    RO[p][k, :] = RO[p][k, :] * wvec[kk]
            pltpu.async_copy(RO[p], acc_sh.at[DD[p]], SC[p], add=True)

        def slot(chunk, p):
            @pl.when(chunk + 1 < cpt)
            def _():
                wait_idx(1 - p)
                @pl.when(chunk >= 1)
                def _():
                    wait_acc(1 - p)
                fire_gathers(1 - p)
            process(chunk, p)
            @pl.when(chunk + 2 < cpt)
            def _():
                fire_idx(chunk + 2, p)

        fire_idx(0, 0)
        wait_idx(0)
        fire_gathers(0)
        fire_idx(1, 1)

        @pl.loop(0, cpt // 2)
        def _pair(t):
            slot(2 * t, 0)
            slot(2 * t + 1, 1)

        wait_den(0)
        wait_den(1)
        wait_acc(0)
        wait_acc(1)
        plsc.subcore_barrier()

        pltpu.sync_copy(acc_sh.at[pl.ds(rbase, rpt)],
                        acc_o.at[pl.ds(hbase + rbase, rpt)])
        pltpu.sync_copy(den_sh.at[pl.ds(rbase, rpt)],
                        den_o.at[pl.ds(hbase + rbase, rpt)])

    return sc_edge


def kernel(x, edge_index, W1, a_src1, a_dst1, b1, W2, a_src2, a_dst2, b2):
    n = x.shape[0]
    e = edge_index.shape[1]
    n2 = ((n + 8 + 127) // 128) * 128
    step = _NS * _K * 2
    e_pad = ((e + step - 1) // step) * step
    grid_b = 4352 if n2 % 4352 == 0 else 128
    g = n2 // grid_b

    xp = jnp.pad(x, ((0, n2 - n), (0, 0)))
    src_p = jnp.concatenate(
        [edge_index[0], jnp.zeros((e_pad - e,), jnp.int32)])
    dst_p = jnp.concatenate(
        [edge_index[1], jnp.full((e_pad - e,), n, jnp.int32)])

    f32 = jnp.float32
    node_vec = jax.ShapeDtypeStruct((2, n2), f32)
    node_mat = jax.ShapeDtypeStruct((2, n2, 16), f32)

    pro1 = pl.pallas_call(
        _pro1_body,
        grid=(g,),
        in_specs=[
            pl.BlockSpec((grid_b, 3), lambda i: (i, 0)),
            pl.BlockSpec((3, 32), lambda i: (0, 0)),
            pl.BlockSpec((2, 16), lambda i: (0, 0)),
            pl.BlockSpec((2, 16), lambda i: (0, 0)),
        ],
        out_specs=[
            pl.BlockSpec((2, grid_b, 16), lambda i: (0, i, 0)),
            pl.BlockSpec((2, grid_b), lambda i: (0, i)),
            pl.BlockSpec((2, grid_b), lambda i: (0, i)),
            pl.BlockSpec((1, 8), lambda i: (0, 0)),
        ],
        out_shape=[node_mat, node_vec, node_vec,
                   jax.ShapeDtypeStruct((1, 8), f32)],
    )
    init_k = pl.pallas_call(
        _init_body,
        grid=(g,),
        in_specs=[
            pl.BlockSpec((2, grid_b, 16), lambda i: (0, i, 0)),
            pl.BlockSpec((2, grid_b), lambda i: (0, i)),
            pl.BlockSpec((2, grid_b), lambda i: (0, i)),
            pl.BlockSpec((1, 8), lambda i: (0, 0)),
        ],
        out_specs=[
            pl.BlockSpec((2, grid_b, 16), lambda i: (0, i, 0)),
            pl.BlockSpec((2, grid_b), lambda i: (0, i)),
            pl.BlockSpec((2, 16), lambda i: (0, 0)),
        ],
        out_shape=[node_mat, node_vec,
                   jax.ShapeDtypeStruct((2, 16), f32)],
    )
    mid = pl.pallas_call(
        _mid_body,
        grid=(g,),
        in_specs=[
            pl.BlockSpec((2, grid_b, 16), lambda i: (0, i, 0)),
            pl.BlockSpec((2, grid_b), lambda i: (0, i)),
            pl.BlockSpec((1, 32), lambda i: (0, 0)),
            pl.BlockSpec((32, 14), lambda i: (0, 0)),
            pl.BlockSpec((2, 7), lambda i: (0, 0)),
            pl.BlockSpec((2, 7), lambda i: (0, 0)),
        ],
        out_specs=[
            pl.BlockSpec((2, grid_b, 16), lambda i: (0, i, 0)),
            pl.BlockSpec((2, grid_b), lambda i: (0, i)),
            pl.BlockSpec((2, grid_b), lambda i: (0, i)),
            pl.BlockSpec((1, 8), lambda i: (0, 0)),
        ],
        out_shape=[node_mat, node_vec, node_vec,
                   jax.ShapeDtypeStruct((1, 8), f32)],
    )
    post2 = pl.pallas_call(
        _post2_body,
        grid=(g,),
        in_specs=[
            pl.BlockSpec((2, grid_b, 16), lambda i: (0, i, 0)),
            pl.BlockSpec((2, grid_b), lambda i: (0, i)),
            pl.BlockSpec((1, 7), lambda i: (0, 0)),
        ],
        out_specs=pl.BlockSpec((grid_b, 7), lambda i: (i, 0)),
        out_shape=jax.ShapeDtypeStruct((n2, 7), f32),
    )
    sc_edge = _make_sc_edge(n2, e_pad)

    def flat(a):
        return a.reshape((-1,) + a.shape[2:])

    h_st, as_st, ad_st, m1 = pro1(xp, W1, a_src1, a_dst1)
    acc0, den0, m16 = init_k(h_st, as_st, ad_st, m1)
    acc1, den1 = sc_edge(src_p, dst_p, flat(h_st), flat(as_st),
                         flat(ad_st), flat(acc0), flat(den0),
                         m16.reshape(-1))
    acc1 = acc1.reshape(2, n2, 16)
    den1 = den1.reshape(2, n2)

    h2_st, as2, ad2, m2 = mid(acc1, den1, b1.reshape(1, 32), W2,
                              a_src2, a_dst2)
    acc02, den02, m216 = init_k(h2_st, as2, ad2, m2)
    acc2, den2 = sc_edge(src_p, dst_p, flat(h2_st), flat(as2), flat(ad2),
                         flat(acc02), flat(den02), m216.reshape(-1))
    acc2 = acc2.reshape(2, n2, 16)
    den2 = den2.reshape(2, n2)

    out = post2(acc2, den2, b2.reshape(1, 7))
    return out[:n]
